# Initial kernel scaffold; baseline (speedup 1.0000x reference)
import jax
import jax.numpy as jnp
from jax.experimental import pallas as pl


def kernel(x, x_res, edge_index, edge_attr, node_deg, W_pre0, b_pre0, W_pre1, b_pre1, emb_table, W_gate, W_value, W_post, b_post, degree_param):
    raise NotImplementedError("write your pallas kernel here")


# speedup vs baseline: 9.0557x; 9.0557x over previous
import functools

import jax
import jax.numpy as jnp
from jax import lax
from jax.experimental import pallas as pl
from jax.experimental.pallas import tpu as pltpu
from jax.experimental.pallas import tpu_sc as plsc

N = 10000
E = 320000
WIDTH = 128
NUM_HEAD = 8
GSIZE = WIDTH // NUM_HEAD
BOND = 33
EPS = 1e-5

_PRE_BLK = 2000
_MSG_BLK = 512
_WIN = 128
_SC_TILES = 16


def _pre_body(x_ref, w0_ref, b0_ref, w1_ref, b1_ref, h0_ref, h1_ref):
    x = x_ref[...]
    h0_ref[...] = jnp.dot(x, w0_ref[...], preferred_element_type=jnp.float32) + b0_ref[...]
    h1_ref[...] = jnp.dot(x, w1_ref[...], preferred_element_type=jnp.float32) + b1_ref[...]


def _msg_body(g0_ref, g1_ref, attr_ref, tab_ref, m_ref, wg_ref, wv_ref, msg_ref):
    g = g0_ref[...] + g1_ref[...]
    m = m_ref[...]
    mu = jnp.dot(g, m, preferred_element_type=jnp.float32)
    msq = jnp.dot(g * g, m, preferred_element_type=jnp.float32)
    var = msq - mu * mu
    xx = (g - mu) * lax.rsqrt(var + EPS)

    attr = attr_ref[...]
    lane = lax.broadcasted_iota(jnp.int32, (_MSG_BLK, WIDTH), 1)
    c = jnp.zeros((_MSG_BLK, WIDTH), jnp.float32)
    for k in range(4):
        ak = attr[:, k][:, None]
        c = c + jnp.where((lane == ak) & (lane > 0), 1.0, 0.0)
    cnt = jnp.sum(c, axis=1, keepdims=True)
    bias = jnp.dot(c, tab_ref[...], preferred_element_type=jnp.float32)
    bias = bias / jnp.maximum(cnt, 1.0)

    gate = jnp.maximum(
        jnp.dot(xx + bias, wg_ref[...], preferred_element_type=jnp.float32), 0.0)
    val = jnp.dot(xx, wv_ref[...], preferred_element_type=jnp.float32)
    msg_ref[...] = gate * val


def _post_body(a0_ref, a1_ref, wp_ref, bp_ref, deg_ref, dp_ref, xres_ref, out_ref):
    agg = a0_ref[...] + a1_ref[...]
    out = jnp.dot(agg, wp_ref[...], preferred_element_type=jnp.float32) + bp_ref[...]
    scale = jnp.exp(dp_ref[...] * jnp.log(deg_ref[...]))
    out_ref[...] = scale * out + xres_ref[...]


_MESH = plsc.VectorSubcoreMesh(core_axis_name="core", subcore_axis_name="subcore")


def _gather(h0, h1, src, dst):
    @functools.partial(
        pl.kernel,
        out_type=(jax.ShapeDtypeStruct((E, WIDTH), jnp.float32),
                  jax.ShapeDtypeStruct((E, WIDTH), jnp.float32)),
        mesh=_MESH)
    def k(h0_hbm, h1_hbm, src_hbm, dst_hbm, g0_hbm, g1_hbm):
        def body(i0, i1, o0, o1):
            pltpu.sync_copy(h0_hbm.at[i0.at[0]], o0)
            pltpu.sync_copy(h1_hbm.at[i1.at[0]], o1)
        pltpu.emit_pipeline(
            body,
            grid=(E // _WIN,),
            in_specs=[pl.BlockSpec((1, _WIN), lambda i: (0, i)),
                      pl.BlockSpec((1, _WIN), lambda i: (0, i))],
            out_specs=[pl.BlockSpec((_WIN, WIDTH), lambda i: (i, 0)),
                       pl.BlockSpec((_WIN, WIDTH), lambda i: (i, 0))],
            core_axis_name=("core", "subcore"),
            dimension_semantics=(pltpu.PARALLEL,),
        )(src_hbm, dst_hbm, g0_hbm, g1_hbm)

    return k(h0, h1, src, dst)


_NPAD = 10240


def _scatter(msg, dst, zeros):
    rows = _NPAD // _SC_TILES

    @functools.partial(
        pl.kernel,
        out_type=jax.ShapeDtypeStruct((2, _NPAD, WIDTH), jnp.float32),
        mesh=_MESH,
        scratch_types=[pltpu.VMEM_SHARED((_NPAD, WIDTH), jnp.float32)])
    def k(msg_hbm, dst_hbm, z_hbm, out_hbm, acc):
        cid = lax.axis_index("core")
        sid = lax.axis_index("subcore")
        pltpu.sync_copy(z_hbm.at[pl.ds(sid * rows, rows)],
                        acc.at[pl.ds(sid * rows, rows)])
        plsc.subcore_barrier()

        def body(m_v, i_v):
            pltpu.sync_copy(m_v, acc.at[i_v.at[0]], add=True)

        pltpu.emit_pipeline(
            body,
            grid=(E // _WIN,),
            in_specs=[pl.BlockSpec((_WIN, WIDTH), lambda i: (i, 0)),
                      pl.BlockSpec((1, _WIN), lambda i: (0, i))],
            out_specs=[],
            core_axis_name=("core", "subcore"),
            dimension_semantics=(pltpu.PARALLEL,),
        )(msg_hbm, dst_hbm)
        plsc.subcore_barrier()
        pltpu.sync_copy(acc.at[pl.ds(sid * rows, rows)],
                        out_hbm.at[cid, pl.ds(sid * rows, rows)])

    return k(msg, dst, zeros)


def kernel(x, x_res, edge_index, edge_attr, node_deg, W_pre0, b_pre0, W_pre1,
           b_pre1, emb_table, W_gate, W_value, W_post, b_post, degree_param):
    f32 = jnp.float32
    src = edge_index[0].reshape(1, E)
    dst = edge_index[1].reshape(1, E)

    wp_specs = [
        pl.BlockSpec((_PRE_BLK, WIDTH), lambda i: (i, 0)),
        pl.BlockSpec((WIDTH, WIDTH), lambda i: (0, 0)),
        pl.BlockSpec((1, WIDTH), lambda i: (0, 0)),
        pl.BlockSpec((WIDTH, WIDTH), lambda i: (0, 0)),
        pl.BlockSpec((1, WIDTH), lambda i: (0, 0)),
    ]
    h0, h1 = pl.pallas_call(
        _pre_body,
        grid=(N // _PRE_BLK,),
        in_specs=wp_specs,
        out_specs=[pl.BlockSpec((_PRE_BLK, WIDTH), lambda i: (i, 0))] * 2,
        out_shape=[jax.ShapeDtypeStruct((N, WIDTH), f32)] * 2,
    )(x, W_pre0.T, b_pre0.reshape(1, WIDTH), W_pre1.T, b_pre1.reshape(1, WIDTH))

    g0, g1 = _gather(h0, h1, src, dst)

    eye8 = jnp.eye(NUM_HEAD, dtype=f32)
    wg_full = jnp.einsum(
        'goc,gh->gcho', W_gate.reshape(NUM_HEAD, GSIZE, GSIZE), eye8
    ).reshape(WIDTH, WIDTH)
    wv_full = jnp.einsum(
        'goc,gh->gcho', W_value.reshape(NUM_HEAD, GSIZE, GSIZE), eye8
    ).reshape(WIDTH, WIDTH)
    m_full = (jnp.einsum('gh,co->gcho', eye8, jnp.ones((GSIZE, GSIZE), f32))
              / GSIZE).reshape(WIDTH, WIDTH)
    tab_pad = jnp.zeros((WIDTH, WIDTH), f32).at[:BOND].set(emb_table)

    msg = pl.pallas_call(
        _msg_body,
        grid=(E // _MSG_BLK,),
        in_specs=[
            pl.BlockSpec((_MSG_BLK, WIDTH), lambda i: (i, 0)),
            pl.BlockSpec((_MSG_BLK, WIDTH), lambda i: (i, 0)),
            pl.BlockSpec((_MSG_BLK, 4), lambda i: (i, 0)),
            pl.BlockSpec((WIDTH, WIDTH), lambda i: (0, 0)),
            pl.BlockSpec((WIDTH, WIDTH), lambda i: (0, 0)),
            pl.BlockSpec((WIDTH, WIDTH), lambda i: (0, 0)),
            pl.BlockSpec((WIDTH, WIDTH), lambda i: (0, 0)),
        ],
        out_specs=pl.BlockSpec((_MSG_BLK, WIDTH), lambda i: (i, 0)),
        out_shape=jax.ShapeDtypeStruct((E, WIDTH), f32),
    )(g0, g1, edge_attr, tab_pad, m_full, wg_full, wv_full)

    aggs = _scatter(msg, dst, jnp.zeros((_NPAD, WIDTH), f32))

    out = pl.pallas_call(
        _post_body,
        grid=(N // _PRE_BLK,),
        in_specs=[
            pl.BlockSpec((_PRE_BLK, WIDTH), lambda i: (i, 0)),
            pl.BlockSpec((_PRE_BLK, WIDTH), lambda i: (i, 0)),
            pl.BlockSpec((WIDTH, WIDTH), lambda i: (0, 0)),
            pl.BlockSpec((1, WIDTH), lambda i: (0, 0)),
            pl.BlockSpec((_PRE_BLK, 1), lambda i: (i, 0)),
            pl.BlockSpec((1, WIDTH), lambda i: (0, 0)),
            pl.BlockSpec((_PRE_BLK, WIDTH), lambda i: (i, 0)),
        ],
        out_specs=pl.BlockSpec((_PRE_BLK, WIDTH), lambda i: (i, 0)),
        out_shape=jax.ShapeDtypeStruct((N, WIDTH), f32),
    )(aggs[0], aggs[1], W_post.T, b_post.reshape(1, WIDTH),
      node_deg.reshape(N, 1), degree_param.reshape(1, WIDTH), x_res)

    return out


# speedup vs baseline: 9.0626x; 1.0008x over previous
import functools

import jax
import jax.numpy as jnp
from jax import lax
from jax.experimental import pallas as pl
from jax.experimental.pallas import tpu as pltpu
from jax.experimental.pallas import tpu_sc as plsc

N = 10000
E = 320000
WIDTH = 128
NUM_HEAD = 8
GSIZE = WIDTH // NUM_HEAD
BOND = 33
EPS = 1e-5

_PRE_BLK = 2000
_MSG_BLK = 512
_WIN = 128
_SC_TILES = 16


def _pre_body(x_ref, w0_ref, b0_ref, w1_ref, b1_ref, h0_ref, h1_ref):
    x = x_ref[...].astype(jnp.bfloat16)
    h0_ref[...] = jnp.dot(x, w0_ref[...], preferred_element_type=jnp.float32) + b0_ref[...]
    h1_ref[...] = jnp.dot(x, w1_ref[...], preferred_element_type=jnp.float32) + b1_ref[...]


def _msg_body(g0_ref, g1_ref, attr_ref, tab_ref, m_ref, wg_ref, wv_ref, msg_ref):
    g = g0_ref[...] + g1_ref[...]
    m = m_ref[...]
    mu = jnp.dot(g.astype(jnp.bfloat16), m, preferred_element_type=jnp.float32)
    msq = jnp.dot((g * g).astype(jnp.bfloat16), m,
                  preferred_element_type=jnp.float32)
    var = msq - mu * mu
    xx = (g - mu) * lax.rsqrt(var + EPS)

    attr = attr_ref[...]
    lane = lax.broadcasted_iota(jnp.int32, (_MSG_BLK, WIDTH), 1)
    c = jnp.zeros((_MSG_BLK, WIDTH), jnp.float32)
    for k in range(4):
        ak = attr[:, k][:, None]
        c = c + jnp.where((lane == ak) & (lane > 0), 1.0, 0.0)
    cnt = jnp.sum(c, axis=1, keepdims=True)
    bias = jnp.dot(c.astype(jnp.bfloat16), tab_ref[...],
                   preferred_element_type=jnp.float32)
    bias = bias / jnp.maximum(cnt, 1.0)

    gate = jnp.maximum(
        jnp.dot((xx + bias).astype(jnp.bfloat16), wg_ref[...],
                preferred_element_type=jnp.float32), 0.0)
    val = jnp.dot(xx.astype(jnp.bfloat16), wv_ref[...],
                  preferred_element_type=jnp.float32)
    msg_ref[...] = gate * val


def _post_body(a0_ref, a1_ref, wp_ref, bp_ref, deg_ref, dp_ref, xres_ref, out_ref):
    agg = a0_ref[...] + a1_ref[...]
    out = jnp.dot(agg.astype(jnp.bfloat16), wp_ref[...],
                  preferred_element_type=jnp.float32) + bp_ref[...]
    scale = jnp.exp(dp_ref[...] * jnp.log(deg_ref[...]))
    out_ref[...] = scale * out + xres_ref[...]


_MESH = plsc.VectorSubcoreMesh(core_axis_name="core", subcore_axis_name="subcore")


def _gather(h0, h1, src, dst):
    @functools.partial(
        pl.kernel,
        out_type=(jax.ShapeDtypeStruct((E, WIDTH), jnp.float32),
                  jax.ShapeDtypeStruct((E, WIDTH), jnp.float32)),
        mesh=_MESH)
    def k(h0_hbm, h1_hbm, src_hbm, dst_hbm, g0_hbm, g1_hbm):
        def body(i0, i1, o0, o1):
            pltpu.sync_copy(h0_hbm.at[i0.at[0]], o0)
            pltpu.sync_copy(h1_hbm.at[i1.at[0]], o1)
        pltpu.emit_pipeline(
            body,
            grid=(E // _WIN,),
            in_specs=[pl.BlockSpec((1, _WIN), lambda i: (0, i)),
                      pl.BlockSpec((1, _WIN), lambda i: (0, i))],
            out_specs=[pl.BlockSpec((_WIN, WIDTH), lambda i: (i, 0)),
                       pl.BlockSpec((_WIN, WIDTH), lambda i: (i, 0))],
            core_axis_name=("core", "subcore"),
            dimension_semantics=(pltpu.PARALLEL,),
        )(src_hbm, dst_hbm, g0_hbm, g1_hbm)

    return k(h0, h1, src, dst)


_NPAD = 10240


def _scatter(msg, dst, zeros):
    rows = _NPAD // _SC_TILES

    @functools.partial(
        pl.kernel,
        out_type=jax.ShapeDtypeStruct((2, _NPAD, WIDTH), jnp.float32),
        mesh=_MESH,
        scratch_types=[pltpu.VMEM_SHARED((_NPAD, WIDTH), jnp.float32)])
    def k(msg_hbm, dst_hbm, z_hbm, out_hbm, acc):
        cid = lax.axis_index("core")
        sid = lax.axis_index("subcore")
        pltpu.sync_copy(z_hbm.at[pl.ds(sid * rows, rows)],
                        acc.at[pl.ds(sid * rows, rows)])
        plsc.subcore_barrier()

        def body(m_v, i_v):
            pltpu.sync_copy(m_v, acc.at[i_v.at[0]], add=True)

        pltpu.emit_pipeline(
            body,
            grid=(E // _WIN,),
            in_specs=[pl.BlockSpec((_WIN, WIDTH), lambda i: (i, 0)),
                      pl.BlockSpec((1, _WIN), lambda i: (0, i))],
            out_specs=[],
            core_axis_name=("core", "subcore"),
            dimension_semantics=(pltpu.PARALLEL,),
        )(msg_hbm, dst_hbm)
        plsc.subcore_barrier()
        pltpu.sync_copy(acc.at[pl.ds(sid * rows, rows)],
                        out_hbm.at[cid, pl.ds(sid * rows, rows)])

    return k(msg, dst, zeros)


def kernel(x, x_res, edge_index, edge_attr, node_deg, W_pre0, b_pre0, W_pre1,
           b_pre1, emb_table, W_gate, W_value, W_post, b_post, degree_param):
    f32 = jnp.float32
    src = edge_index[0].reshape(1, E)
    dst = edge_index[1].reshape(1, E)

    wp_specs = [
        pl.BlockSpec((_PRE_BLK, WIDTH), lambda i: (i, 0)),
        pl.BlockSpec((WIDTH, WIDTH), lambda i: (0, 0)),
        pl.BlockSpec((1, WIDTH), lambda i: (0, 0)),
        pl.BlockSpec((WIDTH, WIDTH), lambda i: (0, 0)),
        pl.BlockSpec((1, WIDTH), lambda i: (0, 0)),
    ]
    h0, h1 = pl.pallas_call(
        _pre_body,
        grid=(N // _PRE_BLK,),
        in_specs=wp_specs,
        out_specs=[pl.BlockSpec((_PRE_BLK, WIDTH), lambda i: (i, 0))] * 2,
        out_shape=[jax.ShapeDtypeStruct((N, WIDTH), f32)] * 2,
    )(x, W_pre0.T, b_pre0.reshape(1, WIDTH), W_pre1.T, b_pre1.reshape(1, WIDTH))

    g0, g1 = _gather(h0, h1, src, dst)

    eye8 = jnp.eye(NUM_HEAD, dtype=f32)
    wg_full = jnp.einsum(
        'goc,gh->gcho', W_gate.reshape(NUM_HEAD, GSIZE, GSIZE), eye8
    ).reshape(WIDTH, WIDTH)
    wv_full = jnp.einsum(
        'goc,gh->gcho', W_value.reshape(NUM_HEAD, GSIZE, GSIZE), eye8
    ).reshape(WIDTH, WIDTH)
    m_full = (jnp.einsum('gh,co->gcho', eye8, jnp.ones((GSIZE, GSIZE), f32))
              / GSIZE).reshape(WIDTH, WIDTH)
    tab_pad = jnp.zeros((WIDTH, WIDTH), f32).at[:BOND].set(emb_table)

    msg = pl.pallas_call(
        _msg_body,
        grid=(E // _MSG_BLK,),
        in_specs=[
            pl.BlockSpec((_MSG_BLK, WIDTH), lambda i: (i, 0)),
            pl.BlockSpec((_MSG_BLK, WIDTH), lambda i: (i, 0)),
            pl.BlockSpec((_MSG_BLK, 4), lambda i: (i, 0)),
            pl.BlockSpec((WIDTH, WIDTH), lambda i: (0, 0)),
            pl.BlockSpec((WIDTH, WIDTH), lambda i: (0, 0)),
            pl.BlockSpec((WIDTH, WIDTH), lambda i: (0, 0)),
            pl.BlockSpec((WIDTH, WIDTH), lambda i: (0, 0)),
        ],
        out_specs=pl.BlockSpec((_MSG_BLK, WIDTH), lambda i: (i, 0)),
        out_shape=jax.ShapeDtypeStruct((E, WIDTH), f32),
    )(g0, g1, edge_attr, tab_pad, m_full, wg_full, wv_full)

    aggs = _scatter(msg, dst, jnp.zeros((_NPAD, WIDTH), f32))

    out = pl.pallas_call(
        _post_body,
        grid=(N // _PRE_BLK,),
        in_specs=[
            pl.BlockSpec((_PRE_BLK, WIDTH), lambda i: (i, 0)),
            pl.BlockSpec((_PRE_BLK, WIDTH), lambda i: (i, 0)),
            pl.BlockSpec((WIDTH, WIDTH), lambda i: (0, 0)),
            pl.BlockSpec((1, WIDTH), lambda i: (0, 0)),
            pl.BlockSpec((_PRE_BLK, 1), lambda i: (i, 0)),
            pl.BlockSpec((1, WIDTH), lambda i: (0, 0)),
            pl.BlockSpec((_PRE_BLK, WIDTH), lambda i: (i, 0)),
        ],
        out_specs=pl.BlockSpec((_PRE_BLK, WIDTH), lambda i: (i, 0)),
        out_shape=jax.ShapeDtypeStruct((N, WIDTH), f32),
    )(aggs[0], aggs[1], W_post.T, b_post.reshape(1, WIDTH),
      node_deg.reshape(N, 1), degree_param.reshape(1, WIDTH), x_res)

    return out


# speedup vs baseline: 13.0649x; 1.4416x over previous
import functools

import jax
import jax.numpy as jnp
from jax import lax
from jax.experimental import pallas as pl
from jax.experimental.pallas import tpu as pltpu
from jax.experimental.pallas import tpu_sc as plsc

N = 10000
E = 320000
WIDTH = 128
NUM_HEAD = 8
GSIZE = WIDTH // NUM_HEAD
BOND = 33
EPS = 1e-5

_PRE_BLK = 2000
_MSG_BLK = 1280
_WIN = 128
_SC_TILES = 16


def _pre_body(x_ref, w0_ref, b0_ref, w1_ref, b1_ref, h0_ref, h1_ref):
    x = x_ref[...].astype(jnp.bfloat16)
    h0_ref[...] = jnp.dot(x, w0_ref[...], preferred_element_type=jnp.float32) + b0_ref[...]
    h1_ref[...] = jnp.dot(x, w1_ref[...], preferred_element_type=jnp.float32) + b1_ref[...]


def _msg_body(g0_ref, g1_ref, attr_ref, tabg_ref, m_ref, wg_ref, wv_ref, msg_ref):
    bf = jnp.bfloat16
    g = g0_ref[...] + g1_ref[...]
    gb = g.astype(bf)
    m = m_ref[...]
    mu = jnp.dot(gb, m, preferred_element_type=jnp.float32)
    msq = jnp.dot(gb * gb, m, preferred_element_type=jnp.float32)
    r = lax.rsqrt(jnp.maximum(msq - mu * mu, 0.0) + EPS)
    xc = (g - mu).astype(bf)
    tg = jnp.dot(xc, wg_ref[...], preferred_element_type=jnp.float32)
    tv = jnp.dot(xc, wv_ref[...], preferred_element_type=jnp.float32)

    attr = attr_ref[...]
    ab = attr.astype(bf)
    laneb = lax.broadcasted_iota(jnp.int32, (_MSG_BLK, WIDTH), 1).astype(bf)
    c = jnp.zeros((_MSG_BLK, WIDTH), bf)
    for k in range(4):
        c = c + jnp.where(laneb == ab[:, k][:, None],
                          jnp.ones((), bf), jnp.zeros((), bf))
    bg = jnp.dot(c, tabg_ref[...], preferred_element_type=jnp.float32)
    cnt = jnp.sum((attr != 0).astype(jnp.float32), axis=1, keepdims=True)
    rc = 1.0 / jnp.maximum(cnt, 1.0)

    gate = jnp.maximum(tg * r + bg * rc, 0.0)
    msg_ref[...] = gate * (tv * r)


def _post_body(a0_ref, a1_ref, wp_ref, bp_ref, deg_ref, dp_ref, xres_ref, out_ref):
    agg = a0_ref[...] + a1_ref[...]
    out = jnp.dot(agg.astype(jnp.bfloat16), wp_ref[...],
                  preferred_element_type=jnp.float32) + bp_ref[...]
    scale = jnp.exp(dp_ref[...] * jnp.log(deg_ref[...]))
    out_ref[...] = scale * out + xres_ref[...]


_MESH = plsc.VectorSubcoreMesh(core_axis_name="core", subcore_axis_name="subcore")


def _gather(h0, h1, src, dst):
    @functools.partial(
        pl.kernel,
        out_type=(jax.ShapeDtypeStruct((E, WIDTH), jnp.float32),
                  jax.ShapeDtypeStruct((E, WIDTH), jnp.float32)),
        mesh=_MESH)
    def k(h0_hbm, h1_hbm, src_hbm, dst_hbm, g0_hbm, g1_hbm):
        def body(i0, i1, o0, o1):
            pltpu.sync_copy(h0_hbm.at[i0.at[0]], o0)
            pltpu.sync_copy(h1_hbm.at[i1.at[0]], o1)
        pltpu.emit_pipeline(
            body,
            grid=(E // _WIN,),
            in_specs=[pl.BlockSpec((1, _WIN), lambda i: (0, i)),
                      pl.BlockSpec((1, _WIN), lambda i: (0, i))],
            out_specs=[pl.BlockSpec((_WIN, WIDTH), lambda i: (i, 0)),
                       pl.BlockSpec((_WIN, WIDTH), lambda i: (i, 0))],
            core_axis_name=("core", "subcore"),
            dimension_semantics=(pltpu.PARALLEL,),
        )(src_hbm, dst_hbm, g0_hbm, g1_hbm)

    return k(h0, h1, src, dst)


_NPAD = 10240


def _scatter(msg, dst, zeros):
    rows = _NPAD // _SC_TILES

    @functools.partial(
        pl.kernel,
        out_type=jax.ShapeDtypeStruct((2, _NPAD, WIDTH), jnp.float32),
        mesh=_MESH,
        scratch_types=[pltpu.VMEM_SHARED((_NPAD, WIDTH), jnp.float32)])
    def k(msg_hbm, dst_hbm, z_hbm, out_hbm, acc):
        cid = lax.axis_index("core")
        sid = lax.axis_index("subcore")
        pltpu.sync_copy(z_hbm.at[pl.ds(sid * rows, rows)],
                        acc.at[pl.ds(sid * rows, rows)])
        plsc.subcore_barrier()

        def body(m_v, i_v):
            pltpu.sync_copy(m_v, acc.at[i_v.at[0]], add=True)

        pltpu.emit_pipeline(
            body,
            grid=(E // _WIN,),
            in_specs=[pl.BlockSpec((_WIN, WIDTH), lambda i: (i, 0)),
                      pl.BlockSpec((1, _WIN), lambda i: (0, i))],
            out_specs=[],
            core_axis_name=("core", "subcore"),
            dimension_semantics=(pltpu.PARALLEL,),
        )(msg_hbm, dst_hbm)
        plsc.subcore_barrier()
        pltpu.sync_copy(acc.at[pl.ds(sid * rows, rows)],
                        out_hbm.at[cid, pl.ds(sid * rows, rows)])

    return k(msg, dst, zeros)


def kernel(x, x_res, edge_index, edge_attr, node_deg, W_pre0, b_pre0, W_pre1,
           b_pre1, emb_table, W_gate, W_value, W_post, b_post, degree_param):
    f32 = jnp.float32
    src = edge_index[0].reshape(1, E)
    dst = edge_index[1].reshape(1, E)

    wp_specs = [
        pl.BlockSpec((_PRE_BLK, WIDTH), lambda i: (i, 0)),
        pl.BlockSpec((WIDTH, WIDTH), lambda i: (0, 0)),
        pl.BlockSpec((1, WIDTH), lambda i: (0, 0)),
        pl.BlockSpec((WIDTH, WIDTH), lambda i: (0, 0)),
        pl.BlockSpec((1, WIDTH), lambda i: (0, 0)),
    ]
    h0, h1 = pl.pallas_call(
        _pre_body,
        grid=(N // _PRE_BLK,),
        in_specs=wp_specs,
        out_specs=[pl.BlockSpec((_PRE_BLK, WIDTH), lambda i: (i, 0))] * 2,
        out_shape=[jax.ShapeDtypeStruct((N, WIDTH), f32)] * 2,
    )(x, W_pre0.T, b_pre0.reshape(1, WIDTH), W_pre1.T, b_pre1.reshape(1, WIDTH))

    g0, g1 = _gather(h0, h1, src, dst)

    eye8 = jnp.eye(NUM_HEAD, dtype=f32)
    wg_full = jnp.einsum(
        'goc,gh->gcho', W_gate.reshape(NUM_HEAD, GSIZE, GSIZE), eye8
    ).reshape(WIDTH, WIDTH)
    wv_full = jnp.einsum(
        'goc,gh->gcho', W_value.reshape(NUM_HEAD, GSIZE, GSIZE), eye8
    ).reshape(WIDTH, WIDTH)
    m_full = (jnp.einsum('gh,co->gcho', eye8, jnp.ones((GSIZE, GSIZE), f32))
              / GSIZE).reshape(WIDTH, WIDTH)
    tab_pad = jnp.zeros((WIDTH, WIDTH), f32).at[:BOND].set(emb_table).at[0].set(0.0)
    tabg = tab_pad @ wg_full
    wg_bf = wg_full.astype(jnp.bfloat16)
    wv_bf = wv_full.astype(jnp.bfloat16)
    m_bf = m_full.astype(jnp.bfloat16)
    tabg_bf = tabg.astype(jnp.bfloat16)

    msg = pl.pallas_call(
        _msg_body,
        grid=(E // _MSG_BLK,),
        in_specs=[
            pl.BlockSpec((_MSG_BLK, WIDTH), lambda i: (i, 0)),
            pl.BlockSpec((_MSG_BLK, WIDTH), lambda i: (i, 0)),
            pl.BlockSpec((_MSG_BLK, 4), lambda i: (i, 0)),
            pl.BlockSpec((WIDTH, WIDTH), lambda i: (0, 0)),
            pl.BlockSpec((WIDTH, WIDTH), lambda i: (0, 0)),
            pl.BlockSpec((WIDTH, WIDTH), lambda i: (0, 0)),
            pl.BlockSpec((WIDTH, WIDTH), lambda i: (0, 0)),
        ],
        out_specs=pl.BlockSpec((_MSG_BLK, WIDTH), lambda i: (i, 0)),
        out_shape=jax.ShapeDtypeStruct((E, WIDTH), f32),
    )(g0, g1, edge_attr, tabg_bf, m_bf, wg_bf, wv_bf)

    aggs = _scatter(msg, dst, jnp.zeros((_NPAD, WIDTH), f32))

    out = pl.pallas_call(
        _post_body,
        grid=(N // _PRE_BLK,),
        in_specs=[
            pl.BlockSpec((_PRE_BLK, WIDTH), lambda i: (i, 0)),
            pl.BlockSpec((_PRE_BLK, WIDTH), lambda i: (i, 0)),
            pl.BlockSpec((WIDTH, WIDTH), lambda i: (0, 0)),
            pl.BlockSpec((1, WIDTH), lambda i: (0, 0)),
            pl.BlockSpec((_PRE_BLK, 1), lambda i: (i, 0)),
            pl.BlockSpec((1, WIDTH), lambda i: (0, 0)),
            pl.BlockSpec((_PRE_BLK, WIDTH), lambda i: (i, 0)),
        ],
        out_specs=pl.BlockSpec((_PRE_BLK, WIDTH), lambda i: (i, 0)),
        out_shape=jax.ShapeDtypeStruct((N, WIDTH), f32),
    )(aggs[0], aggs[1], W_post.T, b_post.reshape(1, WIDTH),
      node_deg.reshape(N, 1), degree_param.reshape(1, WIDTH), x_res)

    return out
